# baseline (speedup 1.0000x reference)
import functools

import jax
import jax.numpy as jnp
from jax import lax
from jax.experimental import pallas as pl
from jax.experimental.pallas import tpu as pltpu
from jax.experimental.pallas import tpu_sc as plsc

_B, _S, _D, _C = 16, 80, 768, 3
_M = 32
_NOFF = _M * (_M - 1)
_PRW = 4 * _NOFF
_HALF = _PRW // 2

_TN = (((0,), (0,)), ((), ()))
_NT = (((1,), (1,)), ((), ()))
_HI = lax.Precision.HIGHEST

_LANE = 16


def _sc_body(hm_hbm, srt_hbm, pr_hbm, hmv, srv, rankv, idxv, s01, prv):
  core = lax.axis_index("c")
  sub = lax.axis_index("s")
  wid = sub * 2 + core
  b = wid % _B
  h = wid // _B

  pltpu.sync_copy(hm_hbm.at[pl.ds(b * _S, _S)], hmv)
  pltpu.sync_copy(srt_hbm, srv)

  iota = lax.iota(jnp.int32, _LANE)
  cand0 = hmv[pl.ds(0, _LANE)]
  cand1 = hmv[pl.ds(_LANE, _LANE)]
  lane0 = iota
  lane1 = iota + _LANE

  def rank_step(j, carry):
    r0, r1 = carry
    spl = plsc.load_gather(hmv, [jnp.full((_LANE,), 0, jnp.int32) + j])
    b0 = (spl > cand0) | ((spl == cand0) & (j < lane0))
    b1 = (spl > cand1) | ((spl == cand1) & (j < lane1))
    return r0 + b0.astype(jnp.int32), r1 + b1.astype(jnp.int32)

  zero = jnp.zeros((_LANE,), jnp.int32)
  r0, r1 = lax.fori_loop(0, _S, rank_step, (zero, zero))
  rankv[pl.ds(0, _LANE)] = r0
  rankv[pl.ds(_LANE, _LANE)] = r1

  def pos_step(j, carry):
    p0, p1 = carry
    spl = plsc.load_gather(rankv, [jnp.full((_LANE,), 0, jnp.int32) + j])
    return p0 + (spl < r0).astype(jnp.int32), p1 + (spl < r1).astype(jnp.int32)

  p0, p1 = lax.fori_loop(0, _M, pos_step, (zero, zero))
  plsc.store_scatter(idxv, [p0], r0)
  plsc.store_scatter(idxv, [p1], r1)

  for g in range(2):
    sel = idxv[pl.ds(g * _LANE, _LANE)]
    s01[pl.ds(g * _LANE, _LANE)] = plsc.load_gather(srv, [sel])
    s01[pl.ds(_M + g * _LANE, _LANE)] = plsc.load_gather(srv, [sel + _S])

  base_t = h * (_NOFF // _LANE // 2)

  def pr_step(t, _):
    q = (base_t + t) * _LANE + iota
    i = q // (_M - 1)
    jj = q % (_M - 1)
    j = jj + (jj >= i).astype(jnp.int32)
    a = plsc.load_gather(s01, [i])
    bb = plsc.load_gather(s01, [i + _M])
    c = plsc.load_gather(s01, [j])
    d = plsc.load_gather(s01, [j + _M])
    f = 4 * q - _HALF * h
    plsc.store_scatter(prv, [f], a)
    plsc.store_scatter(prv, [f + 1], bb)
    plsc.store_scatter(prv, [f + 2], c)
    plsc.store_scatter(prv, [f + 3], d)
    return 0

  lax.fori_loop(0, _NOFF // _LANE // 2, pr_step, 0)
  pltpu.sync_copy(prv, pr_hbm.at[pl.ds(b * _PRW + h * _HALF, _HALF)])


def _sc_select(hm, srt_flat):
  mesh = plsc.VectorSubcoreMesh(
      core_axis_name="c", subcore_axis_name="s", num_cores=2, num_subcores=16)
  return pl.kernel(
      _sc_body,
      out_type=jax.ShapeDtypeStruct((_B * _PRW,), jnp.int32),
      mesh=mesh,
      compiler_params=pltpu.CompilerParams(needs_layout_passes=False),
      scratch_types=[
          pltpu.VMEM((_S,), jnp.float32),
          pltpu.VMEM((2 * _S,), jnp.int32),
          pltpu.VMEM((_M,), jnp.int32),
          pltpu.VMEM((_M,), jnp.int32),
          pltpu.VMEM((2 * _M,), jnp.int32),
          pltpu.VMEM((_HALF,), jnp.int32),
      ],
  )(hm, srt_flat)


def _tc_body(x_ref, hm_ref, w1_ref, w2_ref, w3t_ref, bpair_ref, logits_ref):
  xb = x_ref[0]
  hm_row = hm_ref[0]

  eye_s = (lax.broadcasted_iota(jnp.int32, (_S, _S), 0) ==
           lax.broadcasted_iota(jnp.int32, (_S, _S), 1)).astype(jnp.float32)
  hm_col = lax.dot_general(eye_s, hm_row, _NT, precision=_HI)

  row_i = lax.broadcasted_iota(jnp.int32, (_S, _S), 0)
  col_j = lax.broadcasted_iota(jnp.int32, (_S, _S), 1)
  beats = (hm_row > hm_col) | ((hm_row == hm_col) & (col_j < row_i))
  rank_col = jnp.sum(beats.astype(jnp.float32), axis=1, keepdims=True)

  r32_col = rank_col[:_M]
  eye_m = (lax.broadcasted_iota(jnp.int32, (_M, _M), 0) ==
           lax.broadcasted_iota(jnp.int32, (_M, _M), 1)).astype(jnp.float32)
  r32_row = lax.dot_general(r32_col, eye_m, _TN, precision=_HI)
  pos_col = jnp.sum((r32_row < r32_col).astype(jnp.float32),
                    axis=1, keepdims=True)
  oh_pos = (pos_col == lax.broadcasted_iota(
      jnp.int32, (_M, _M), 1).astype(jnp.float32)).astype(jnp.float32)
  oh_rank = (r32_col == lax.broadcasted_iota(
      jnp.int32, (_M, _S), 1).astype(jnp.float32)).astype(jnp.float32)
  P = lax.dot_general(oh_pos, oh_rank, _TN, precision=_HI)

  x_rk = jnp.dot(P, xb, precision=_HI)

  A = jnp.dot(x_rk, w1_ref[...], precision=_HI)
  Bt = lax.dot_general(w2_ref[...], x_rk, (((0,), (1,)), ((), ())),
                       precision=_HI)
  bp = bpair_ref[...]
  sig = []
  for c in range(_C):
    wc = w3t_ref[c:c + 1, :]
    Mc = lax.dot_general(x_rk * wc, x_rk, _NT, precision=_HI)
    pre = Mc + A[:, c:c + 1] + Bt[c:c + 1, :] + bp[0, c]
    sig.append(jax.nn.sigmoid(pre))
  mx = jnp.maximum(jnp.maximum(sig[0], sig[1]), sig[2])
  es = [jnp.exp(s - mx) for s in sig]
  den = es[0] + es[1] + es[2]
  for c in range(_C):
    logits_ref[0, c] = es[c] / den


@jax.jit
def _run(x, span_ranges, W_span, b_span, W_pair, b_pair):
  W1 = W_pair[:_D, :]
  W2 = W_pair[_D:2 * _D, :]
  W3T = W_pair[2 * _D:, :].T
  bpair = b_pair.reshape(1, _C)
  srt_flat = jnp.ravel(span_ranges.T)
  hm = jax.nn.sigmoid(x @ W_span + b_span).mean(axis=-1)

  pr_flat = _sc_select(jnp.ravel(hm), srt_flat)

  logits_full = pl.pallas_call(
      _tc_body,
      grid=(_B,),
      in_specs=[
          pl.BlockSpec((1, _S, _D), lambda b: (b, 0, 0)),
          pl.BlockSpec((1, 1, _S), lambda b: (b, 0, 0)),
          pl.BlockSpec((_D, _C), lambda b: (0, 0)),
          pl.BlockSpec((_D, _C), lambda b: (0, 0)),
          pl.BlockSpec((_C, _D), lambda b: (0, 0)),
          pl.BlockSpec((1, _C), lambda b: (0, 0)),
      ],
      out_specs=pl.BlockSpec((1, _C, _M, _M), lambda b: (b, 0, 0, 0)),
      out_shape=jax.ShapeDtypeStruct((_B, _C, _M, _M), jnp.float32),
  )(x, hm.reshape(_B, 1, _S), W1, W2, W3T, bpair)

  logits = logits_full.reshape(_B, _C, _M * _M).transpose(0, 2, 1)
  logits = logits[:, 1:, :].reshape(_B, _M - 1, _M + 1, _C)[:, :, :_M, :]
  logits = logits.reshape(_B, _NOFF, _C)
  pair_ranges = pr_flat.reshape(_B, _NOFF, 2, 2)
  return logits, pair_ranges


def kernel(x, span_ranges, W_span, b_span, W_pair, b_pair):
  return _run(x, span_ranges, W_span, b_span, W_pair, b_pair)


# speedup vs baseline: 3.3336x; 3.3336x over previous
import jax
import jax.numpy as jnp
from jax import lax
from jax.experimental import pallas as pl

_B, _S, _D, _C = 16, 80, 768, 3
_M = 32
_NOFF = _M * (_M - 1)

_TN = (((0,), (0,)), ((), ()))
_NT = (((1,), (1,)), ((), ()))
_HI = lax.Precision.HIGHEST


def _tc_body(x_ref, hmr_ref, hmc_ref, sr_ref, srt_ref, w1_ref, w2t_ref,
             w3t_ref, bpair_ref, logits_ref, pr_ref):
  xb = x_ref[0]
  hm_row = hmr_ref[0]
  hm_col = hmc_ref[0]

  row_i = lax.broadcasted_iota(jnp.int32, (_S, _S), 0)
  col_j = lax.broadcasted_iota(jnp.int32, (_S, _S), 1)
  beats = ((hm_row > hm_col) |
           ((hm_row == hm_col) & (col_j < row_i))).astype(jnp.float32)
  rank_col = jnp.sum(beats, axis=1, keepdims=True)
  rank_row = (_S - 1) - jnp.sum(beats, axis=0, keepdims=True)

  r32_col = rank_col[:_M]
  r32_row = rank_row[:, :_M]
  pos_col = jnp.sum((r32_row < r32_col).astype(jnp.float32),
                    axis=1, keepdims=True)
  oh_pos = (pos_col == lax.broadcasted_iota(
      jnp.int32, (_M, _M), 1).astype(jnp.float32)).astype(jnp.float32)
  oh_rank = (r32_col == lax.broadcasted_iota(
      jnp.int32, (_M, _S), 1).astype(jnp.float32)).astype(jnp.float32)
  P = lax.dot_general(oh_pos, oh_rank, _TN)
  PT = lax.dot_general(oh_rank, oh_pos, _TN)

  x_rk = jnp.dot(P, xb, precision=_HI)

  A = jnp.dot(x_rk, w1_ref[...])
  Bt = lax.dot_general(w2t_ref[...], x_rk, _NT)
  bp = bpair_ref[...]
  sig = []
  for c in range(_C):
    wc = w3t_ref[c:c + 1, :]
    Mc = lax.dot_general(x_rk * wc, x_rk, _NT)
    pre = Mc + A[:, c:c + 1] + Bt[c:c + 1, :] + bp[0, c]
    sig.append(jax.nn.sigmoid(pre))
  mx = jnp.maximum(jnp.maximum(sig[0], sig[1]), sig[2])
  es = [jnp.exp(s - mx) for s in sig]
  den = es[0] + es[1] + es[2]
  for c in range(_C):
    logits_ref[0, c] = es[c] / den

  Pi = P.astype(jnp.int32)
  PTi = PT.astype(jnp.int32)
  sr0 = jnp.sum(Pi * srt_ref[0:1, :], axis=1, keepdims=True)
  sr1 = jnp.sum(Pi * srt_ref[1:2, :], axis=1, keepdims=True)
  sr0_row = jnp.sum(PTi * sr_ref[:, 0:1], axis=0, keepdims=True)
  sr1_row = jnp.sum(PTi * sr_ref[:, 1:2], axis=0, keepdims=True)
  zero_m = jnp.zeros((_M, _M), jnp.int32)
  pr_ref[0, 0] = zero_m + sr0
  pr_ref[0, 1] = zero_m + sr1
  pr_ref[0, 2] = zero_m + sr0_row
  pr_ref[0, 3] = zero_m + sr1_row


import functools


@functools.partial(jax.jit, static_argnames=("interpret",))
def _run(x, span_ranges, W_span, b_span, W_pair, b_pair, interpret=False):
  srT = span_ranges.T
  W1 = W_pair[:_D, :]
  W2T = W_pair[_D:2 * _D, :].T
  W3T = W_pair[2 * _D:, :].T
  bpair = b_pair.reshape(1, _C)
  hm = jax.nn.sigmoid(x @ W_span + b_span).mean(axis=-1)
  hmr = hm.reshape(_B, 1, _S)
  hmc = hm.reshape(_B, _S, 1)

  logits_full, pr_full = pl.pallas_call(
      _tc_body,
      grid=(_B,),
      in_specs=[
          pl.BlockSpec((1, _S, _D), lambda b: (b, 0, 0)),
          pl.BlockSpec((1, 1, _S), lambda b: (b, 0, 0)),
          pl.BlockSpec((1, _S, 1), lambda b: (b, 0, 0)),
          pl.BlockSpec((_S, 2), lambda b: (0, 0)),
          pl.BlockSpec((2, _S), lambda b: (0, 0)),
          pl.BlockSpec((_D, _C), lambda b: (0, 0)),
          pl.BlockSpec((_C, _D), lambda b: (0, 0)),
          pl.BlockSpec((_C, _D), lambda b: (0, 0)),
          pl.BlockSpec((1, _C), lambda b: (0, 0)),
      ],
      out_specs=[
          pl.BlockSpec((1, _C, _M, _M), lambda b: (b, 0, 0, 0)),
          pl.BlockSpec((1, 4, _M, _M), lambda b: (b, 0, 0, 0)),
      ],
      out_shape=[
          jax.ShapeDtypeStruct((_B, _C, _M, _M), jnp.float32),
          jax.ShapeDtypeStruct((_B, 4, _M, _M), jnp.int32),
      ],
      interpret=interpret,
  )(x, hmr, hmc, span_ranges, srT, W1, W2T, W3T, bpair)

  logits = logits_full.reshape(_B, _C, _M * _M).transpose(0, 2, 1)
  logits = logits[:, 1:, :].reshape(_B, _M - 1, _M + 1, _C)[:, :, :_M, :]
  logits = logits.reshape(_B, _NOFF, _C)
  pr = pr_full.reshape(_B, 4, _M * _M).transpose(0, 2, 1)
  pr = pr[:, 1:, :].reshape(_B, _M - 1, _M + 1, 4)[:, :, :_M, :]
  pr = pr.reshape(_B, _NOFF, 2, 2)
  return logits, pr


def kernel(x, span_ranges, W_span, b_span, W_pair, b_pair):
  return _run(x, span_ranges, W_span, b_span, W_pair, b_pair)


# speedup vs baseline: 4.0719x; 1.2214x over previous
import jax
import jax.numpy as jnp
from jax import lax
from jax.experimental import pallas as pl

_B, _S, _D, _C = 16, 80, 768, 3
_M = 32
_NOFF = _M * (_M - 1)

_TN = (((0,), (0,)), ((), ()))
_NT = (((1,), (1,)), ((), ()))
_HI = lax.Precision.HIGHEST


_BS = 4


def _tc_body(x_ref, hmr_ref, hmc_ref, sr_ref, srt_ref, w1_ref, w2t_ref,
             w3t_ref, bpair_ref, logits_ref, pr_ref):
  for e in range(_BS):
    _tc_one(e, x_ref, hmr_ref, hmc_ref, sr_ref, srt_ref, w1_ref, w2t_ref,
            w3t_ref, bpair_ref, logits_ref, pr_ref)


def _tc_one(e, x_ref, hmr_ref, hmc_ref, sr_ref, srt_ref, w1_ref, w2t_ref,
            w3t_ref, bpair_ref, logits_ref, pr_ref):
  xb = x_ref[e]
  hm_row = hmr_ref[e]
  hm_col = hmc_ref[e]

  row_i = lax.broadcasted_iota(jnp.int32, (_S, _S), 0)
  col_j = lax.broadcasted_iota(jnp.int32, (_S, _S), 1)
  beats = ((hm_row > hm_col) |
           ((hm_row == hm_col) & (col_j < row_i))).astype(jnp.float32)
  rank_col = jnp.sum(beats, axis=1, keepdims=True)
  rank_row = (_S - 1) - jnp.sum(beats, axis=0, keepdims=True)

  r32_col = rank_col[:_M]
  r32_row = rank_row[:, :_M]
  pos_col = jnp.sum((r32_row < r32_col).astype(jnp.float32),
                    axis=1, keepdims=True)
  oh_pos = (pos_col == lax.broadcasted_iota(
      jnp.int32, (_M, _M), 1).astype(jnp.float32)).astype(jnp.float32)
  oh_rank = (r32_col == lax.broadcasted_iota(
      jnp.int32, (_M, _S), 1).astype(jnp.float32)).astype(jnp.float32)
  P = lax.dot_general(oh_pos, oh_rank, _TN)
  PT = lax.dot_general(oh_rank, oh_pos, _TN)

  x_rk = jnp.dot(P, xb, precision=_HI)

  A = jnp.dot(x_rk, w1_ref[...])
  Bt = lax.dot_general(w2t_ref[...], x_rk, _NT)
  bp = bpair_ref[...]
  sig = []
  for c in range(_C):
    wc = w3t_ref[c:c + 1, :]
    Mc = lax.dot_general(x_rk * wc, x_rk, _NT)
    pre = Mc + A[:, c:c + 1] + Bt[c:c + 1, :] + bp[0, c]
    sig.append(jax.nn.sigmoid(pre))
  mx = jnp.maximum(jnp.maximum(sig[0], sig[1]), sig[2])
  es = [jnp.exp(s - mx) for s in sig]
  den = es[0] + es[1] + es[2]
  for c in range(_C):
    logits_ref[e, c] = es[c] / den

  Pi = P.astype(jnp.int32)
  PTi = PT.astype(jnp.int32)
  sr0 = jnp.sum(Pi * srt_ref[0:1, :], axis=1, keepdims=True)
  sr1 = jnp.sum(Pi * srt_ref[1:2, :], axis=1, keepdims=True)
  sr0_row = jnp.sum(PTi * sr_ref[:, 0:1], axis=0, keepdims=True)
  sr1_row = jnp.sum(PTi * sr_ref[:, 1:2], axis=0, keepdims=True)
  zero_m = jnp.zeros((_M, _M), jnp.int32)
  pr_ref[e, 0] = zero_m + sr0
  pr_ref[e, 1] = zero_m + sr1
  pr_ref[e, 2] = zero_m + sr0_row
  pr_ref[e, 3] = zero_m + sr1_row


import functools


@functools.partial(jax.jit, static_argnames=("interpret",))
def _run(x, span_ranges, W_span, b_span, W_pair, b_pair, interpret=False):
  srT = span_ranges.T
  W1 = W_pair[:_D, :]
  W2T = W_pair[_D:2 * _D, :].T
  W3T = W_pair[2 * _D:, :].T
  bpair = b_pair.reshape(1, _C)
  hm = jax.nn.sigmoid(x @ W_span + b_span).mean(axis=-1)
  hmr = hm.reshape(_B, 1, _S)
  hmc = hm.reshape(_B, _S, 1)

  logits_full, pr_full = pl.pallas_call(
      _tc_body,
      grid=(_B // _BS,),
      in_specs=[
          pl.BlockSpec((_BS, _S, _D), lambda b: (b, 0, 0)),
          pl.BlockSpec((_BS, 1, _S), lambda b: (b, 0, 0)),
          pl.BlockSpec((_BS, _S, 1), lambda b: (b, 0, 0)),
          pl.BlockSpec((_S, 2), lambda b: (0, 0)),
          pl.BlockSpec((2, _S), lambda b: (0, 0)),
          pl.BlockSpec((_D, _C), lambda b: (0, 0)),
          pl.BlockSpec((_C, _D), lambda b: (0, 0)),
          pl.BlockSpec((_C, _D), lambda b: (0, 0)),
          pl.BlockSpec((1, _C), lambda b: (0, 0)),
      ],
      out_specs=[
          pl.BlockSpec((_BS, _C, _M, _M), lambda b: (b, 0, 0, 0)),
          pl.BlockSpec((_BS, 4, _M, _M), lambda b: (b, 0, 0, 0)),
      ],
      out_shape=[
          jax.ShapeDtypeStruct((_B, _C, _M, _M), jnp.float32),
          jax.ShapeDtypeStruct((_B, 4, _M, _M), jnp.int32),
      ],
      interpret=interpret,
  )(x, hmr, hmc, span_ranges, srT, W1, W2T, W3T, bpair)

  logits = logits_full.reshape(_B, _C, _M * _M).transpose(0, 2, 1)
  logits = logits[:, 1:, :].reshape(_B, _M - 1, _M + 1, _C)[:, :, :_M, :]
  logits = logits.reshape(_B, _NOFF, _C)
  pr = pr_full.reshape(_B, 4, _M * _M).transpose(0, 2, 1)
  pr = pr[:, 1:, :].reshape(_B, _M - 1, _M + 1, 4)[:, :, :_M, :]
  pr = pr.reshape(_B, _NOFF, 2, 2)
  return logits, pr


def kernel(x, span_ranges, W_span, b_span, W_pair, b_pair):
  return _run(x, span_ranges, W_span, b_span, W_pair, b_pair)


# speedup vs baseline: 4.1637x; 1.0225x over previous
import jax
import jax.numpy as jnp
from jax import lax
from jax.experimental import pallas as pl

_B, _S, _D, _C = 16, 80, 768, 3
_M = 32
_NOFF = _M * (_M - 1)

_TN = (((0,), (0,)), ((), ()))
_NT = (((1,), (1,)), ((), ()))
_HI = lax.Precision.HIGHEST


_BS = 8


def _tc_body(x_ref, hmr_ref, hmc_ref, sr_ref, srt_ref, w1_ref, w2t_ref,
             w3t_ref, bpair_ref, logits_ref, pr_ref):
  for e in range(_BS):
    _tc_one(e, x_ref, hmr_ref, hmc_ref, sr_ref, srt_ref, w1_ref, w2t_ref,
            w3t_ref, bpair_ref, logits_ref, pr_ref)


def _tc_one(e, x_ref, hmr_ref, hmc_ref, sr_ref, srt_ref, w1_ref, w2t_ref,
            w3t_ref, bpair_ref, logits_ref, pr_ref):
  xb = x_ref[e]
  hm_row = hmr_ref[e]
  hm_col = hmc_ref[e]

  row_i = lax.broadcasted_iota(jnp.int32, (_S, _S), 0)
  col_j = lax.broadcasted_iota(jnp.int32, (_S, _S), 1)
  beats = ((hm_row > hm_col) |
           ((hm_row == hm_col) & (col_j < row_i))).astype(jnp.float32)
  rank_col = jnp.sum(beats, axis=1, keepdims=True)
  rank_row = (_S - 1) - jnp.sum(beats, axis=0, keepdims=True)

  r32_col = rank_col[:_M]
  r32_row = rank_row[:, :_M]
  pos_col = jnp.sum((r32_row < r32_col).astype(jnp.float32),
                    axis=1, keepdims=True)
  oh_pos = (pos_col == lax.broadcasted_iota(
      jnp.int32, (_M, _M), 1).astype(jnp.float32)).astype(jnp.float32)
  oh_rank = (r32_col == lax.broadcasted_iota(
      jnp.int32, (_M, _S), 1).astype(jnp.float32)).astype(jnp.float32)
  P = lax.dot_general(oh_pos, oh_rank, _TN)
  PT = lax.dot_general(oh_rank, oh_pos, _TN)

  x_rk = jnp.dot(P, xb, precision=_HI)

  A = jnp.dot(x_rk, w1_ref[...])
  Bt = lax.dot_general(w2t_ref[...], x_rk, _NT)
  bp = bpair_ref[...]
  sig = []
  for c in range(_C):
    wc = w3t_ref[c:c + 1, :]
    Mc = lax.dot_general(x_rk * wc, x_rk, _NT)
    pre = Mc + A[:, c:c + 1] + Bt[c:c + 1, :] + bp[0, c]
    sig.append(jax.nn.sigmoid(pre))
  mx = jnp.maximum(jnp.maximum(sig[0], sig[1]), sig[2])
  es = [jnp.exp(s - mx) for s in sig]
  den = es[0] + es[1] + es[2]
  for c in range(_C):
    logits_ref[e, c] = es[c] / den

  Pi = P.astype(jnp.int32)
  PTi = PT.astype(jnp.int32)
  sr0 = jnp.sum(Pi * srt_ref[0:1, :], axis=1, keepdims=True)
  sr1 = jnp.sum(Pi * srt_ref[1:2, :], axis=1, keepdims=True)
  sr0_row = jnp.sum(PTi * sr_ref[:, 0:1], axis=0, keepdims=True)
  sr1_row = jnp.sum(PTi * sr_ref[:, 1:2], axis=0, keepdims=True)
  zero_m = jnp.zeros((_M, _M), jnp.int32)
  pr_ref[e, 0] = zero_m + sr0
  pr_ref[e, 1] = zero_m + sr1
  pr_ref[e, 2] = zero_m + sr0_row
  pr_ref[e, 3] = zero_m + sr1_row


import functools


@functools.partial(jax.jit, static_argnames=("interpret",))
def _run(x, span_ranges, W_span, b_span, W_pair, b_pair, interpret=False):
  srT = span_ranges.T
  W1 = W_pair[:_D, :]
  W2T = W_pair[_D:2 * _D, :].T
  W3T = W_pair[2 * _D:, :].T
  bpair = b_pair.reshape(1, _C)
  hm = jax.nn.sigmoid(x @ W_span + b_span).mean(axis=-1)
  hmr = hm.reshape(_B, 1, _S)
  hmc = hm.reshape(_B, _S, 1)

  logits_full, pr_full = pl.pallas_call(
      _tc_body,
      grid=(_B // _BS,),
      in_specs=[
          pl.BlockSpec((_BS, _S, _D), lambda b: (b, 0, 0)),
          pl.BlockSpec((_BS, 1, _S), lambda b: (b, 0, 0)),
          pl.BlockSpec((_BS, _S, 1), lambda b: (b, 0, 0)),
          pl.BlockSpec((_S, 2), lambda b: (0, 0)),
          pl.BlockSpec((2, _S), lambda b: (0, 0)),
          pl.BlockSpec((_D, _C), lambda b: (0, 0)),
          pl.BlockSpec((_C, _D), lambda b: (0, 0)),
          pl.BlockSpec((_C, _D), lambda b: (0, 0)),
          pl.BlockSpec((1, _C), lambda b: (0, 0)),
      ],
      out_specs=[
          pl.BlockSpec((_BS, _C, _M, _M), lambda b: (b, 0, 0, 0)),
          pl.BlockSpec((_BS, 4, _M, _M), lambda b: (b, 0, 0, 0)),
      ],
      out_shape=[
          jax.ShapeDtypeStruct((_B, _C, _M, _M), jnp.float32),
          jax.ShapeDtypeStruct((_B, 4, _M, _M), jnp.int32),
      ],
      interpret=interpret,
  )(x, hmr, hmc, span_ranges, srT, W1, W2T, W3T, bpair)

  logits = logits_full.reshape(_B, _C, _M * _M).transpose(0, 2, 1)
  logits = logits[:, 1:, :].reshape(_B, _M - 1, _M + 1, _C)[:, :, :_M, :]
  logits = logits.reshape(_B, _NOFF, _C)
  pr = pr_full.reshape(_B, 4, _M * _M).transpose(0, 2, 1)
  pr = pr[:, 1:, :].reshape(_B, _M - 1, _M + 1, 4)[:, :, :_M, :]
  pr = pr.reshape(_B, _NOFF, 2, 2)
  return logits, pr


def kernel(x, span_ranges, W_span, b_span, W_pair, b_pair):
  return _run(x, span_ranges, W_span, b_span, W_pair, b_pair)
